# baseline (speedup 1.0000x reference)
import functools

import jax
import jax.numpy as jnp
from jax import lax
from jax.experimental import pallas as pl
from jax.experimental.pallas import tpu as pltpu
from jax.experimental.pallas import tpu_sc as plsc

N = 10000
E = 320000
C = 128
NC = 2
NS = 16
NW = NC * NS
EPW = E // NW
CHUNK = 80
NCHUNK = EPW // CHUNK
N_PAD = 10240
ROWS_PER_TILE = N_PAD // NS
CPROWS = 64
SUPER = 5
MACRO = SUPER * CHUNK
NM = EPW // MACRO
IROWS = EPW // CHUNK
S = 5
ESL = E // S

_f32 = jnp.float32


def _sc_mesh():
    return plsc.VectorSubcoreMesh(
        core_axis_name="c", subcore_axis_name="s", num_cores=NC, num_subcores=NS
    )


@functools.lru_cache(maxsize=None)
def _make_gather2(D, esl=E):
    epw = esl // NW
    nm = epw // MACRO
    irows = epw // CHUNK

    @functools.partial(
        pl.kernel,
        mesh=_sc_mesh(),
        out_type=(
            jax.ShapeDtypeStruct((esl, D), _f32),
            jax.ShapeDtypeStruct((esl, D), _f32),
        ),
        scratch_types=[
            pltpu.VMEM((irows, CHUNK), jnp.int32),
            pltpu.VMEM((irows, CHUNK), jnp.int32),
            pltpu.VMEM((MACRO, D), _f32),
            pltpu.VMEM((MACRO, D), _f32),
            pltpu.SemaphoreType.DMA,
            pltpu.SemaphoreType.DMA,
            pltpu.SemaphoreType.DMA,
            pltpu.SemaphoreType.DMA,
            pltpu.SemaphoreType.DMA,
        ],
        compiler_params=pltpu.CompilerParams(use_tc_tiling_on_sc=False),
        name=f"sc_gather2_{D}",
    )
    def k(tab_s, tab_r, send2, recv2, gs_h, gr_h,
          idxs, idxr, buf0, buf1, spre, sg0, sg1, sw0, sw1):
        cid = lax.axis_index("c")
        sid = lax.axis_index("s")
        wid = cid * NS + sid
        rbase = wid * irows
        ebase = wid * epw

        def g_start(t, idxall, tab, buf, sem):
            for kk in range(SUPER):
                pltpu.async_copy(
                    tab.at[idxall.at[t * SUPER + kk]],
                    buf.at[pl.ds(kk * CHUNK, CHUNK)], sem)

        def g_wait(buf, sem):
            pltpu.make_async_copy(gs_h.at[pl.ds(ebase, MACRO)], buf, sem).wait()

        def w_start(buf, out_h, t, sem):
            pltpu.async_copy(buf, out_h.at[pl.ds(ebase + t * MACRO, MACRO)], sem)

        def w_wait(buf, sem):
            pltpu.make_async_copy(buf, gs_h.at[pl.ds(ebase, MACRO)], sem).wait()

        pltpu.async_copy(send2.at[pl.ds(rbase, irows)], idxs, spre)
        pltpu.async_copy(recv2.at[pl.ds(rbase, irows)], idxr, spre)
        pltpu.make_async_copy(send2.at[pl.ds(rbase, irows)], idxs, spre).wait()
        pltpu.make_async_copy(recv2.at[pl.ds(rbase, irows)], idxr, spre).wait()

        g_start(0, idxs, tab_s, buf0, sg0)

        def body(t, carry):
            g_wait(buf0, sg0)

            @pl.when(t > 0)
            def _():
                w_wait(buf1, sw1)

            g_start(t, idxr, tab_r, buf1, sg1)
            w_start(buf0, gs_h, t, sw0)
            g_wait(buf1, sg1)
            w_start(buf1, gr_h, t, sw1)

            @pl.when(t + 1 < nm)
            def _():
                w_wait(buf0, sw0)
                g_start(t + 1, idxs, tab_s, buf0, sg0)

            return carry

        lax.fori_loop(0, nm, body, 0)
        w_wait(buf0, sw0)
        w_wait(buf1, sw1)

    return k


@functools.lru_cache(maxsize=None)
def _make_segsum(esl=E):
    epw = esl // NW
    nchunk = epw // CHUNK

    @functools.partial(
        pl.kernel,
        mesh=_sc_mesh(),
        out_type=jax.ShapeDtypeStruct((NC, N_PAD, C), _f32),
        scratch_types=[
            pltpu.VMEM((epw // CHUNK, CHUNK), jnp.int32),
            pltpu.VMEM((CHUNK, C), _f32),
            pltpu.VMEM((CHUNK, C), _f32),
            pltpu.VMEM((CPROWS, C), _f32),
            pltpu.VMEM_SHARED((N_PAD, C), _f32),
            pltpu.SemaphoreType.DMA,
            pltpu.SemaphoreType.DMA,
            pltpu.SemaphoreType.DMA,
            pltpu.SemaphoreType.DMA,
            pltpu.SemaphoreType.DMA,
        ],
        compiler_params=pltpu.CompilerParams(use_tc_tiling_on_sc=False),
        name="sc_segsum",
    )
    def k(e_h, recv2, prev_h, out_h, idxall, buf0, buf1, cp_v, acc_sh,
          spre, sl0, sl1, ss0, ss1):
        cid = lax.axis_index("c")
        sid = lax.axis_index("s")
        wid = cid * NS + sid
        irows = epw // CHUNK
        rbase = wid * irows
        ebase = wid * epw
        r0 = sid * ROWS_PER_TILE

        pltpu.async_copy(recv2.at[pl.ds(rbase, irows)], idxall, spre)

        for kk in range(ROWS_PER_TILE // CPROWS):
            pltpu.sync_copy(
                prev_h.at[cid, pl.ds(r0 + kk * CPROWS, CPROWS)], cp_v)
            pltpu.sync_copy(cp_v, acc_sh.at[pl.ds(r0 + kk * CPROWS, CPROWS)])
        pltpu.make_async_copy(recv2.at[pl.ds(rbase, irows)], idxall, spre).wait()
        plsc.subcore_barrier()

        def ld_start(t, ebuf, sem):
            pltpu.async_copy(e_h.at[pl.ds(ebase + t * CHUNK, CHUNK)], ebuf, sem)

        def ld_wait(ebuf, sem):
            pltpu.make_async_copy(
                e_h.at[pl.ds(ebase, CHUNK)], ebuf, sem).wait()

        def s_start(t, ebuf, sem):
            pltpu.async_copy(ebuf, acc_sh.at[idxall.at[t]], sem, add=True)

        def s_wait(ebuf, sem):
            pltpu.make_async_copy(
                ebuf, acc_sh.at[pl.ds(0, CHUNK)], sem).wait()

        ld_start(0, buf0, sl0)

        def body(t, carry):
            a = 2 * t
            ld_wait(buf0, sl0)

            @pl.when(t > 0)
            def _():
                s_wait(buf1, ss1)

            ld_start(a + 1, buf1, sl1)
            s_start(a, buf0, ss0)
            ld_wait(buf1, sl1)
            s_wait(buf0, ss0)
            ld_start(a + 2, buf0, sl0)
            s_start(a + 1, buf1, ss1)
            return carry

        lax.fori_loop(0, nchunk // 2, body, 0)
        ld_wait(buf0, sl0)
        s_wait(buf1, ss1)
        s_start(nchunk - 1, buf0, ss0)
        s_wait(buf0, ss0)

        plsc.subcore_barrier()
        for kk in range(ROWS_PER_TILE // CPROWS):
            pltpu.sync_copy(acc_sh.at[pl.ds(r0 + kk * CPROWS, CPROWS)], cp_v)
            pltpu.sync_copy(cp_v, out_h.at[cid, pl.ds(r0 + kk * CPROWS, CPROWS)])

    return k


def _dot(a, b):
    return jnp.dot(a, b, preferred_element_type=_f32)


def _bdot(a, b):
    return jnp.dot(a.astype(jnp.bfloat16), b.astype(jnp.bfloat16),
                   preferred_element_type=_f32)


def _ln(z, g, b):
    mu = jnp.mean(z, axis=-1, keepdims=True)
    zc = z - mu
    var = jnp.mean(zc * zc, axis=-1, keepdims=True)
    return zc / jnp.sqrt(var + 1e-5) * g + b


_RND = 0x8000
_HIM = 0xFFFF0000


def _pack_bf16(a):
    u_lo = jax.lax.bitcast_convert_type(a[:, 0:64], jnp.uint32)
    u_hi = jax.lax.bitcast_convert_type(a[:, 64:128], jnp.uint32)
    packed = ((u_lo + jnp.uint32(_RND)) >> 16) | (
        (u_hi + jnp.uint32(_RND)) & jnp.uint32(_HIM))
    return jax.lax.bitcast_convert_type(packed, _f32)


def _unpack_bf16(p):
    u = jax.lax.bitcast_convert_type(p, jnp.uint32)
    lo = jax.lax.bitcast_convert_type(u << 16, _f32)
    hi = jax.lax.bitcast_convert_type(u & jnp.uint32(_HIM), _f32)
    return jnp.concatenate([lo, hi], axis=1)


BE = 2560
EGRID = E // BE


def _edge_enc_body(xs, xr, wd, w2, b1, w2t, b2, w3t, b3, g, bl, out):
    d = xs[...] - xr[...]
    col = lax.broadcasted_iota(jnp.int32, d.shape, 1)
    sq = jnp.where((col >= 4) & (col < 7), d * d, 0.0)
    e2 = jnp.sqrt(jnp.sum(sq, axis=-1, keepdims=True))
    z = _dot(d, wd[...]) + e2 * w2[...] + b1[...]
    z = jax.nn.relu(z)
    z = jax.nn.relu(_dot(z, w2t[...]) + b2[...])
    z = _dot(z, w3t[...]) + b3[...]
    out[...] = _ln(z, g[...], bl[...])


def _edge_encoder(xs, xr, weights):
    ne_ = xs.shape[0]
    dspec = pl.BlockSpec((BE, 16), lambda i: (i, 0))
    wspecs = [pl.BlockSpec(w.shape, lambda i: (0, 0)) for w in weights]
    return pl.pallas_call(
        _edge_enc_body,
        grid=(ne_ // BE,),
        in_specs=[dspec, dspec] + wspecs,
        out_specs=pl.BlockSpec((BE, C), lambda i: (i, 0)),
        out_shape=jax.ShapeDtypeStruct((ne_, C), _f32),
    )(xs, xr, *weights)


def _edge_upd_body(gs, gr, e, we, b1, w2t, b2, w3t, b3, g, bl, out):
    z = gs[...] + gr[...] + _dot(e[...], we[...]) + b1[...]
    z = jax.nn.relu(z)
    z = jax.nn.relu(_dot(z, w2t[...]) + b2[...])
    z = _dot(z, w3t[...]) + b3[...]
    out[...] = e[...] + _ln(z, g[...], bl[...])


def _edge_update(gs, gr, e, weights):
    ne_ = gs.shape[0]
    dspec = pl.BlockSpec((BE, C), lambda i: (i, 0))
    wspecs = [pl.BlockSpec(w.shape, lambda i: (0, 0)) for w in weights]
    return pl.pallas_call(
        _edge_upd_body,
        grid=(ne_ // BE,),
        in_specs=[dspec, dspec, dspec] + wspecs,
        out_specs=pl.BlockSpec((BE, C), lambda i: (i, 0)),
        out_shape=jax.ShapeDtypeStruct((ne_, C), _f32),
    )(gs, gr, e, *weights)


def _node_enc_body(x, w1t, b1, w2t, b2, w3t, b3, g, bl, wst, wrt,
                   h_o, as_o, ar_o):
    z = jax.nn.relu(_dot(x[...], w1t[...]) + b1[...])
    z = jax.nn.relu(_dot(z, w2t[...]) + b2[...])
    z = _dot(z, w3t[...]) + b3[...]
    h = _ln(z, g[...], bl[...])
    h_o[...] = h
    as_o[...] = _dot(h, wst[...])
    ar_o[...] = _dot(h, wrt[...])


def _node_encoder(x, weights):
    return pl.pallas_call(
        _node_enc_body,
        out_shape=(
            jax.ShapeDtypeStruct((N, C), _f32),
            jax.ShapeDtypeStruct((N, C), _f32),
            jax.ShapeDtypeStruct((N, C), _f32),
        ),
    )(x, *weights)


def _node_upd_mid_body(h, p0, p1, v1h, v1a, b1, w2t, b2, w3t, b3, g, bl,
                       wst, wrt, h_o, as_o, ar_o):
    agg = p0[0:N, :] + p1[0:N, :]
    z = jax.nn.relu(_dot(h[...], v1h[...]) + _dot(agg, v1a[...]) + b1[...])
    z = jax.nn.relu(_dot(z, w2t[...]) + b2[...])
    z = _dot(z, w3t[...]) + b3[...]
    hn = h[...] + _ln(z, g[...], bl[...])
    h_o[...] = hn
    as_o[...] = _dot(hn, wst[...])
    ar_o[...] = _dot(hn, wrt[...])


def _node_update_mid(h, p0, p1, weights):
    return pl.pallas_call(
        _node_upd_mid_body,
        out_shape=(
            jax.ShapeDtypeStruct((N, C), _f32),
            jax.ShapeDtypeStruct((N, C), _f32),
            jax.ShapeDtypeStruct((N, C), _f32),
        ),
    )(h, p0, p1, *weights)


def _node_upd_last_body(h, p0, p1, v1h, v1a, b1, w2t, b2, w3t, b3, g, bl,
                        d1t, db1, d2t, db2, d3t, db3, out):
    agg = p0[0:N, :] + p1[0:N, :]
    z = jax.nn.relu(_dot(h[...], v1h[...]) + _dot(agg, v1a[...]) + b1[...])
    z = jax.nn.relu(_dot(z, w2t[...]) + b2[...])
    z = _dot(z, w3t[...]) + b3[...]
    hn = h[...] + _ln(z, g[...], bl[...])
    y = jax.nn.relu(_dot(hn, d1t[...]) + db1[...])
    y = jax.nn.relu(_dot(y, d2t[...]) + db2[...])
    out[...] = _dot(y, d3t[...]) + db3[...]


def _node_update_last(h, p0, p1, weights):
    return pl.pallas_call(
        _node_upd_last_body,
        out_shape=jax.ShapeDtypeStruct((N, 4), _f32),
    )(h, p0, p1, *weights)


def _r(v):
    return v.reshape(1, -1)


def _mlp3(p):
    Ws, bs = p["W"], p["b"]
    return [Ws[0].T, _r(bs[0]), Ws[1].T, _r(bs[1]), Ws[2].T, _r(bs[2])]


def _ln_w(p):
    return [_r(p["ln"]["g"]), _r(p["ln"]["b"])]


def kernel(x, edge_index, pos, params):
    send = edge_index[0]
    recv = edge_index[1]

    enc_e = params["edge_encoder"]
    W1e = enc_e["W"][0]
    wd = jnp.zeros((16, C), _f32)
    wd = wd.at[0:4, :].set(W1e[:, 4:8].T)
    wd = wd.at[4:7, :].set(W1e[:, 0:3].T)
    w2 = _r(W1e[:, 3])
    ee_w = [wd, w2, _r(enc_e["b"][0]), enc_e["W"][1].T, _r(enc_e["b"][1]),
            enc_e["W"][2].T, _r(enc_e["b"][2])] + _ln_w(enc_e)

    layer_w = []
    proj_w = []
    for lp in params["layers"]:
        eu = lp["edge_updater"]
        W1 = eu["W"][0]
        proj_w.append((W1[:, 0:C].T, W1[:, C:2 * C].T))
        eu_w = [W1[:, 2 * C:].T, _r(eu["b"][0]), eu["W"][1].T, _r(eu["b"][1]),
                eu["W"][2].T, _r(eu["b"][2])] + _ln_w(eu)
        nu = lp["node_updater"]
        V1 = nu["W"][0]
        nu_w = [V1[:, 0:C].T, V1[:, C:].T, _r(nu["b"][0]), nu["W"][1].T,
                _r(nu["b"][1]), nu["W"][2].T, _r(nu["b"][2])] + _ln_w(nu)
        layer_w.append((eu_w, nu_w))

    ne = params["node_encoder"]
    ne_w = _mlp3(ne) + _ln_w(ne)

    dec = params["node_decoder"]
    dec_w = [dec["W"][0].T, _r(dec["b"][0]), dec["W"][1].T, _r(dec["b"][1]),
             dec["W"][2].T, _r(dec["b"][2])]

    xp = jnp.concatenate([x, pos, jnp.zeros((N, 9), _f32)], axis=1)
    send2 = send.reshape(E // CHUNK, CHUNK)
    recv2 = recv.reshape(E // CHUNK, CHUNK)
    irows_s = ESL // CHUNK
    s2 = [send2[s * irows_s:(s + 1) * irows_s] for s in range(S)]
    r2 = [recv2[s * irows_s:(s + 1) * irows_s] for s in range(S)]

    e_parts = []
    for s in range(S):
        xs, xr = _make_gather2(16, ESL)(xp, xp, s2[s], r2[s])
        e_parts.append(_edge_encoder(xs, xr, ee_w))

    h, a_s, a_r = _node_encoder(x, ne_w + [proj_w[0][0], proj_w[0][1]])

    zeros_nc = jnp.zeros((NC, N_PAD, C), _f32)
    for i in range(len(layer_w)):
        eu_w, nu_w = layer_w[i]
        new_e = []
        for s in range(S):
            gs, gr = _make_gather2(C, ESL)(a_s, a_r, s2[s], r2[s])
            new_e.append(_edge_update(gs, gr, e_parts[s], eu_w))
        parts = zeros_nc
        for s in range(S):
            parts = _make_segsum(ESL)(new_e[s], r2[s], parts)
        e_parts = new_e
        if i + 1 < len(layer_w):
            h, a_s, a_r = _node_update_mid(
                h, parts[0], parts[1],
                nu_w + [proj_w[i + 1][0], proj_w[i + 1][1]])
        else:
            out = _node_update_last(h, parts[0], parts[1], nu_w + dec_w)
    return out


# speedup vs baseline: 1.0387x; 1.0387x over previous
import functools

import jax
import jax.numpy as jnp
from jax import lax
from jax.experimental import pallas as pl
from jax.experimental.pallas import tpu as pltpu
from jax.experimental.pallas import tpu_sc as plsc

N = 10000
E = 320000
C = 128
NC = 2
NS = 16
NW = NC * NS
EPW = E // NW
CHUNK = 80
NCHUNK = EPW // CHUNK
N_PAD = 10240
ROWS_PER_TILE = N_PAD // NS
CPROWS = 128
SUPER = 5
MACRO = SUPER * CHUNK
NM = EPW // MACRO
IROWS = EPW // CHUNK
S = 5
ESL = E // S

_f32 = jnp.float32


def _sc_mesh():
    return plsc.VectorSubcoreMesh(
        core_axis_name="c", subcore_axis_name="s", num_cores=NC, num_subcores=NS
    )


@functools.lru_cache(maxsize=None)
def _make_gather2(D, esl=E):
    epw = esl // NW
    nm = epw // MACRO
    irows = epw // CHUNK

    @functools.partial(
        pl.kernel,
        mesh=_sc_mesh(),
        out_type=(
            jax.ShapeDtypeStruct((esl, D), _f32),
            jax.ShapeDtypeStruct((esl, D), _f32),
        ),
        scratch_types=[
            pltpu.VMEM((SUPER, CHUNK), jnp.int32),
            pltpu.VMEM((SUPER, CHUNK), jnp.int32),
            pltpu.VMEM((MACRO, D), _f32),
            pltpu.VMEM((MACRO, D), _f32),
            pltpu.SemaphoreType.DMA,
            pltpu.SemaphoreType.DMA,
            pltpu.SemaphoreType.DMA,
            pltpu.SemaphoreType.DMA,
            pltpu.SemaphoreType.DMA,
            pltpu.SemaphoreType.DMA,
        ],
        compiler_params=pltpu.CompilerParams(use_tc_tiling_on_sc=False),
        name=f"sc_gather2_{D}",
    )
    def k(tab_s, tab_r, send2, recv2, gs_h, gr_h,
          idx0, idx1, buf0, buf1, si0, si1, sg0, sg1, sw0, sw1):
        cid = lax.axis_index("c")
        sid = lax.axis_index("s")
        wid = cid * NS + sid
        rbase = wid * irows
        ebase = wid * epw

        def idx_start(t, src2, idxbuf, sem):
            return pltpu.async_copy(
                src2.at[pl.ds(rbase + t * SUPER, SUPER)], idxbuf, sem)

        def idx_wait(src2, idxbuf, sem):
            pltpu.make_async_copy(
                src2.at[pl.ds(rbase, SUPER)], idxbuf, sem).wait()

        def g_start(idxbuf, tab, buf, sem):
            for kk in range(SUPER):
                pltpu.async_copy(
                    tab.at[idxbuf.at[kk]], buf.at[pl.ds(kk * CHUNK, CHUNK)],
                    sem)

        def g_wait(buf, sem):
            pltpu.make_async_copy(gs_h.at[pl.ds(ebase, MACRO)], buf, sem).wait()

        def w_start(buf, out_h, t, sem):
            pltpu.async_copy(buf, out_h.at[pl.ds(ebase + t * MACRO, MACRO)], sem)

        def w_wait(buf, sem):
            pltpu.make_async_copy(buf, gs_h.at[pl.ds(ebase, MACRO)], sem).wait()

        idx_start(0, send2, idx0, si0)
        idx_wait(send2, idx0, si0)
        g_start(idx0, tab_s, buf0, sg0)
        idx_start(0, recv2, idx1, si1)

        def body(t, carry):
            g_wait(buf0, sg0)

            @pl.when(t > 0)
            def _():
                w_wait(buf1, sw1)

            idx_wait(recv2, idx1, si1)
            g_start(idx1, tab_r, buf1, sg1)
            w_start(buf0, gs_h, t, sw0)

            @pl.when(t + 1 < nm)
            def _():
                idx_start(t + 1, send2, idx0, si0)

            g_wait(buf1, sg1)
            w_start(buf1, gr_h, t, sw1)

            @pl.when(t + 1 < nm)
            def _():
                w_wait(buf0, sw0)
                idx_wait(send2, idx0, si0)
                g_start(idx0, tab_s, buf0, sg0)
                idx_start(t + 1, recv2, idx1, si1)

            return carry

        lax.fori_loop(0, nm, body, 0)
        w_wait(buf0, sw0)
        w_wait(buf1, sw1)

    return k


@functools.lru_cache(maxsize=None)
def _make_segsum(esl=E):
    epw = esl // NW
    nchunk = epw // CHUNK

    @functools.partial(
        pl.kernel,
        mesh=_sc_mesh(),
        out_type=jax.ShapeDtypeStruct((NC, N_PAD, C), _f32),
        scratch_types=[
            pltpu.VMEM((1, CHUNK), jnp.int32),
            pltpu.VMEM((1, CHUNK), jnp.int32),
            pltpu.VMEM((CHUNK, C), _f32),
            pltpu.VMEM((CHUNK, C), _f32),
            pltpu.VMEM((CPROWS, C), _f32),
            pltpu.VMEM_SHARED((N_PAD, C), _f32),
            pltpu.SemaphoreType.DMA,
            pltpu.SemaphoreType.DMA,
            pltpu.SemaphoreType.DMA,
            pltpu.SemaphoreType.DMA,
        ],
        name="sc_segsum",
    )
    def k(e_h, recv_h, prev_h, out_h, idx0, idx1, buf0, buf1, cp_v, acc_sh,
          sl0, sl1, ss0, ss1):
        cid = lax.axis_index("c")
        sid = lax.axis_index("s")
        wid = cid * NS + sid
        ebase = wid * epw
        r0 = sid * ROWS_PER_TILE

        pltpu.sync_copy(prev_h.at[cid, pl.ds(r0, ROWS_PER_TILE)],
                        acc_sh.at[pl.ds(r0, ROWS_PER_TILE)])
        plsc.subcore_barrier()

        def ld_start(t, idxbuf, ebuf, sem):
            off = ebase + t * CHUNK
            pltpu.async_copy(
                recv_h.at[pl.ds(off, CHUNK)], idxbuf.at[0], sem)
            pltpu.async_copy(e_h.at[pl.ds(off, CHUNK)], ebuf, sem)

        def ld_wait(idxbuf, ebuf, sem):
            pltpu.make_async_copy(
                recv_h.at[pl.ds(ebase, CHUNK)], idxbuf.at[0], sem).wait()
            pltpu.make_async_copy(
                e_h.at[pl.ds(ebase, CHUNK)], ebuf, sem).wait()

        def s_start(idxbuf, ebuf, sem):
            pltpu.async_copy(ebuf, acc_sh.at[idxbuf.at[0]], sem, add=True)

        def s_wait(ebuf, sem):
            pltpu.make_async_copy(
                ebuf, acc_sh.at[pl.ds(0, CHUNK)], sem).wait()

        ld_start(0, idx0, buf0, sl0)

        def body(t, carry):
            a = 2 * t
            ld_wait(idx0, buf0, sl0)

            @pl.when(t > 0)
            def _():
                s_wait(buf1, ss1)

            ld_start(a + 1, idx1, buf1, sl1)
            s_start(idx0, buf0, ss0)
            ld_wait(idx1, buf1, sl1)
            s_wait(buf0, ss0)
            ld_start(a + 2, idx0, buf0, sl0)
            s_start(idx1, buf1, ss1)
            return carry

        lax.fori_loop(0, nchunk // 2, body, 0)
        ld_wait(idx0, buf0, sl0)
        s_wait(buf1, ss1)
        s_start(idx0, buf0, ss0)
        s_wait(buf0, ss0)

        plsc.subcore_barrier()
        pltpu.sync_copy(acc_sh.at[pl.ds(r0, ROWS_PER_TILE)],
                        out_h.at[cid, pl.ds(r0, ROWS_PER_TILE)])

    return k


def _dot(a, b):
    return jnp.dot(a, b, preferred_element_type=_f32)


def _bdot(a, b):
    return jnp.dot(a.astype(jnp.bfloat16), b.astype(jnp.bfloat16),
                   preferred_element_type=_f32)


def _ln(z, g, b):
    mu = jnp.mean(z, axis=-1, keepdims=True)
    zc = z - mu
    var = jnp.mean(zc * zc, axis=-1, keepdims=True)
    return zc / jnp.sqrt(var + 1e-5) * g + b


_RND = 0x8000
_HIM = 0xFFFF0000


def _pack_bf16(a):
    u_lo = jax.lax.bitcast_convert_type(a[:, 0:64], jnp.uint32)
    u_hi = jax.lax.bitcast_convert_type(a[:, 64:128], jnp.uint32)
    packed = ((u_lo + jnp.uint32(_RND)) >> 16) | (
        (u_hi + jnp.uint32(_RND)) & jnp.uint32(_HIM))
    return jax.lax.bitcast_convert_type(packed, _f32)


def _unpack_bf16(p):
    u = jax.lax.bitcast_convert_type(p, jnp.uint32)
    lo = jax.lax.bitcast_convert_type(u << 16, _f32)
    hi = jax.lax.bitcast_convert_type(u & jnp.uint32(_HIM), _f32)
    return jnp.concatenate([lo, hi], axis=1)


BE = 2560
EGRID = E // BE


def _edge_enc_body(xs, xr, wd, w2, b1, w2t, b2, w3t, b3, g, bl, out):
    d = xs[...] - xr[...]
    col = lax.broadcasted_iota(jnp.int32, d.shape, 1)
    sq = jnp.where((col >= 4) & (col < 7), d * d, 0.0)
    e2 = jnp.sqrt(jnp.sum(sq, axis=-1, keepdims=True))
    z = _dot(d, wd[...]) + e2 * w2[...] + b1[...]
    z = jax.nn.relu(z)
    z = jax.nn.relu(_dot(z, w2t[...]) + b2[...])
    z = _dot(z, w3t[...]) + b3[...]
    out[...] = _ln(z, g[...], bl[...])


def _edge_encoder(xs, xr, weights):
    ne_ = xs.shape[0]
    dspec = pl.BlockSpec((BE, 16), lambda i: (i, 0))
    wspecs = [pl.BlockSpec(w.shape, lambda i: (0, 0)) for w in weights]
    return pl.pallas_call(
        _edge_enc_body,
        grid=(ne_ // BE,),
        in_specs=[dspec, dspec] + wspecs,
        out_specs=pl.BlockSpec((BE, C), lambda i: (i, 0)),
        out_shape=jax.ShapeDtypeStruct((ne_, C), _f32),
    )(xs, xr, *weights)


def _edge_upd_body(gs, gr, e, we, b1, w2t, b2, w3t, b3, g, bl, out):
    z = gs[...] + gr[...] + _dot(e[...], we[...]) + b1[...]
    z = jax.nn.relu(z)
    z = jax.nn.relu(_dot(z, w2t[...]) + b2[...])
    z = _dot(z, w3t[...]) + b3[...]
    out[...] = e[...] + _ln(z, g[...], bl[...])


def _edge_update(gs, gr, e, weights):
    ne_ = gs.shape[0]
    dspec = pl.BlockSpec((BE, C), lambda i: (i, 0))
    wspecs = [pl.BlockSpec(w.shape, lambda i: (0, 0)) for w in weights]
    return pl.pallas_call(
        _edge_upd_body,
        grid=(ne_ // BE,),
        in_specs=[dspec, dspec, dspec] + wspecs,
        out_specs=pl.BlockSpec((BE, C), lambda i: (i, 0)),
        out_shape=jax.ShapeDtypeStruct((ne_, C), _f32),
    )(gs, gr, e, *weights)


def _node_enc_body(x, w1t, b1, w2t, b2, w3t, b3, g, bl, wst, wrt,
                   h_o, as_o, ar_o):
    z = jax.nn.relu(_dot(x[...], w1t[...]) + b1[...])
    z = jax.nn.relu(_dot(z, w2t[...]) + b2[...])
    z = _dot(z, w3t[...]) + b3[...]
    h = _ln(z, g[...], bl[...])
    h_o[...] = h
    as_o[...] = _dot(h, wst[...])
    ar_o[...] = _dot(h, wrt[...])


def _node_encoder(x, weights):
    return pl.pallas_call(
        _node_enc_body,
        out_shape=(
            jax.ShapeDtypeStruct((N, C), _f32),
            jax.ShapeDtypeStruct((N, C), _f32),
            jax.ShapeDtypeStruct((N, C), _f32),
        ),
    )(x, *weights)


def _node_upd_mid_body(h, p0, p1, v1h, v1a, b1, w2t, b2, w3t, b3, g, bl,
                       wst, wrt, h_o, as_o, ar_o):
    agg = p0[0:N, :] + p1[0:N, :]
    z = jax.nn.relu(_dot(h[...], v1h[...]) + _dot(agg, v1a[...]) + b1[...])
    z = jax.nn.relu(_dot(z, w2t[...]) + b2[...])
    z = _dot(z, w3t[...]) + b3[...]
    hn = h[...] + _ln(z, g[...], bl[...])
    h_o[...] = hn
    as_o[...] = _dot(hn, wst[...])
    ar_o[...] = _dot(hn, wrt[...])


def _node_update_mid(h, p0, p1, weights):
    return pl.pallas_call(
        _node_upd_mid_body,
        out_shape=(
            jax.ShapeDtypeStruct((N, C), _f32),
            jax.ShapeDtypeStruct((N, C), _f32),
            jax.ShapeDtypeStruct((N, C), _f32),
        ),
    )(h, p0, p1, *weights)


def _node_upd_last_body(h, p0, p1, v1h, v1a, b1, w2t, b2, w3t, b3, g, bl,
                        d1t, db1, d2t, db2, d3t, db3, out):
    agg = p0[0:N, :] + p1[0:N, :]
    z = jax.nn.relu(_dot(h[...], v1h[...]) + _dot(agg, v1a[...]) + b1[...])
    z = jax.nn.relu(_dot(z, w2t[...]) + b2[...])
    z = _dot(z, w3t[...]) + b3[...]
    hn = h[...] + _ln(z, g[...], bl[...])
    y = jax.nn.relu(_dot(hn, d1t[...]) + db1[...])
    y = jax.nn.relu(_dot(y, d2t[...]) + db2[...])
    out[...] = _dot(y, d3t[...]) + db3[...]


def _node_update_last(h, p0, p1, weights):
    return pl.pallas_call(
        _node_upd_last_body,
        out_shape=jax.ShapeDtypeStruct((N, 4), _f32),
    )(h, p0, p1, *weights)


def _r(v):
    return v.reshape(1, -1)


def _mlp3(p):
    Ws, bs = p["W"], p["b"]
    return [Ws[0].T, _r(bs[0]), Ws[1].T, _r(bs[1]), Ws[2].T, _r(bs[2])]


def _ln_w(p):
    return [_r(p["ln"]["g"]), _r(p["ln"]["b"])]


def kernel(x, edge_index, pos, params):
    send = edge_index[0]
    recv = edge_index[1]

    enc_e = params["edge_encoder"]
    W1e = enc_e["W"][0]
    wd = jnp.zeros((16, C), _f32)
    wd = wd.at[0:4, :].set(W1e[:, 4:8].T)
    wd = wd.at[4:7, :].set(W1e[:, 0:3].T)
    w2 = _r(W1e[:, 3])
    ee_w = [wd, w2, _r(enc_e["b"][0]), enc_e["W"][1].T, _r(enc_e["b"][1]),
            enc_e["W"][2].T, _r(enc_e["b"][2])] + _ln_w(enc_e)

    layer_w = []
    proj_w = []
    for lp in params["layers"]:
        eu = lp["edge_updater"]
        W1 = eu["W"][0]
        proj_w.append((W1[:, 0:C].T, W1[:, C:2 * C].T))
        eu_w = [W1[:, 2 * C:].T, _r(eu["b"][0]), eu["W"][1].T, _r(eu["b"][1]),
                eu["W"][2].T, _r(eu["b"][2])] + _ln_w(eu)
        nu = lp["node_updater"]
        V1 = nu["W"][0]
        nu_w = [V1[:, 0:C].T, V1[:, C:].T, _r(nu["b"][0]), nu["W"][1].T,
                _r(nu["b"][1]), nu["W"][2].T, _r(nu["b"][2])] + _ln_w(nu)
        layer_w.append((eu_w, nu_w))

    ne = params["node_encoder"]
    ne_w = _mlp3(ne) + _ln_w(ne)

    dec = params["node_decoder"]
    dec_w = [dec["W"][0].T, _r(dec["b"][0]), dec["W"][1].T, _r(dec["b"][1]),
             dec["W"][2].T, _r(dec["b"][2])]

    xp = jnp.concatenate([x, pos, jnp.zeros((N, 9), _f32)], axis=1)
    send2 = send.reshape(E // CHUNK, CHUNK)
    recv2 = recv.reshape(E // CHUNK, CHUNK)
    irows_s = ESL // CHUNK
    s2 = [send2[s * irows_s:(s + 1) * irows_s] for s in range(S)]
    r2 = [recv2[s * irows_s:(s + 1) * irows_s] for s in range(S)]
    r1 = [recv[s * ESL:(s + 1) * ESL] for s in range(S)]

    e_parts = []
    for s in range(S):
        xs, xr = _make_gather2(16, ESL)(xp, xp, s2[s], r2[s])
        e_parts.append(_edge_encoder(xs, xr, ee_w))

    h, a_s, a_r = _node_encoder(x, ne_w + [proj_w[0][0], proj_w[0][1]])

    zeros_nc = jnp.zeros((NC, N_PAD, C), _f32)
    for i in range(len(layer_w)):
        eu_w, nu_w = layer_w[i]
        new_e = []
        for s in range(S):
            gs, gr = _make_gather2(C, ESL)(a_s, a_r, s2[s], r2[s])
            new_e.append(_edge_update(gs, gr, e_parts[s], eu_w))
        parts = zeros_nc
        for s in range(S):
            parts = _make_segsum(ESL)(new_e[s], r1[s], parts)
        e_parts = new_e
        if i + 1 < len(layer_w):
            h, a_s, a_r = _node_update_mid(
                h, parts[0], parts[1],
                nu_w + [proj_w[i + 1][0], proj_w[i + 1][1]])
        else:
            out = _node_update_last(h, parts[0], parts[1], nu_w + dec_w)
    return out
